# baseline (speedup 1.0000x reference)
import functools

import jax
import jax.numpy as jnp
from jax import lax
from jax.experimental import pallas as pl
from jax.experimental.pallas import tpu as pltpu
from jax.experimental.pallas import tpu_sc as plsc

B, NA, NB, D = 8, 1000, 1000, 256
NW = 32
PAD = 8192
ROWS_PER_W = PAD // NW
CHUNK = 64
NCH = ROWS_PER_W // CHUNK
LAN = D // 16


def _tc_argmax_body(thr_ref, iou_ref, idx_ref, mask_ref):
    x = iou_ref[0]
    col = lax.broadcasted_iota(jnp.int32, (NA, NB), 1)
    mx = jnp.max(x, axis=1, keepdims=True)
    cand = jnp.where(x == mx, col, NB)
    jst = jnp.min(cand, axis=1, keepdims=True)
    b = pl.program_id(0)
    idx_ref[...] = (jst + b * NB).reshape(1, NA, 1)
    mask_ref[...] = (mx >= thr_ref[0]).astype(jnp.float32).reshape(1, NA, 1)


def _sc_dot_body(ap_hbm, az_hbm, bp_hbm, bz_hbm, idx_hbm,
                 da_hbm, db_hbm, nap_hbm, naz_hbm, ngp_hbm, ngz_hbm,
                 idx_v, ap_v, az_v, gp_v, gz_v,
                 da_v, db_v, nap_v, naz_v, ngp_v, ngz_v, s1, s2):
    wid = lax.axis_index("s") * 2 + lax.axis_index("c")
    for c in range(NCH):
        row0 = wid * ROWS_PER_W + c * CHUNK
        g = wid * NCH + c

        @pl.when(row0 < B * NA)
        def _():
            pltpu.sync_copy(idx_hbm.at[pl.ds(g, 1)], idx_v)
            cp1 = pltpu.async_copy(bp_hbm.at[idx_v.at[0]], gp_v, s1)
            cp2 = pltpu.async_copy(bz_hbm.at[idx_v.at[0]], gz_v, s2)
            pltpu.sync_copy(ap_hbm.at[pl.ds(row0, CHUNK)], ap_v)
            pltpu.sync_copy(az_hbm.at[pl.ds(row0, CHUNK)], az_v)
            cp1.wait()
            cp2.wait()

            def body(r, carry):
                da = jnp.zeros((16,), jnp.float32)
                db = jnp.zeros((16,), jnp.float32)
                nap = jnp.zeros((16,), jnp.float32)
                naz = jnp.zeros((16,), jnp.float32)
                ngp = jnp.zeros((16,), jnp.float32)
                ngz = jnp.zeros((16,), jnp.float32)
                for d in range(LAN):
                    sl = pl.ds(16 * d, 16)
                    ap = ap_v[r, sl]
                    az = az_v[r, sl]
                    gp = gp_v[r, sl]
                    gz = gz_v[r, sl]
                    da = da + ap * gz
                    db = db + gp * az
                    nap = nap + ap * ap
                    naz = naz + az * az
                    ngp = ngp + gp * gp
                    ngz = ngz + gz * gz
                da_v[r, :] = da
                db_v[r, :] = db
                nap_v[r, :] = nap
                naz_v[r, :] = naz
                ngp_v[r, :] = ngp
                ngz_v[r, :] = ngz
                return carry

            lax.fori_loop(0, CHUNK, body, 0, unroll=False)
            sl_out = pl.ds(row0, CHUNK)
            pltpu.sync_copy(da_v, da_hbm.at[sl_out])
            pltpu.sync_copy(db_v, db_hbm.at[sl_out])
            pltpu.sync_copy(nap_v, nap_hbm.at[sl_out])
            pltpu.sync_copy(naz_v, naz_hbm.at[sl_out])
            pltpu.sync_copy(ngp_v, ngp_hbm.at[sl_out])
            pltpu.sync_copy(ngz_v, ngz_hbm.at[sl_out])


def _tc_final_body(da_ref, db_ref, nap_ref, naz_ref, ngp_ref, ngz_ref, m_ref,
                   sa_ref, sb_ref, c_ref):
    eps = jnp.float32(1e-12)

    def nrm(ref):
        return jnp.maximum(jnp.sqrt(jnp.sum(ref[...], axis=1, keepdims=True)), eps)

    da = jnp.sum(da_ref[...], axis=1, keepdims=True)
    db = jnp.sum(db_ref[...], axis=1, keepdims=True)
    cos_a = da / (nrm(nap_ref) * nrm(ngz_ref))
    cos_b = db / (nrm(ngp_ref) * nrm(naz_ref))
    m = m_ref[...].reshape(B * NA, 1)
    sa_ref[...] = jnp.broadcast_to(jnp.sum(m * cos_a), (8, 128))
    sb_ref[...] = jnp.broadcast_to(jnp.sum(m * cos_b), (8, 128))
    cnt = jnp.sum(m.reshape(B, NA, 1), axis=1)
    c_ref[...] = jnp.broadcast_to(cnt, (B, 128))


def kernel(feat_a_p, feat_a_z, feat_b_p, feat_b_z, iou, iou_threshold):
    thr = jnp.asarray(iou_threshold, jnp.float32).reshape(1)

    flat_idx, mask = pl.pallas_call(
        _tc_argmax_body,
        grid=(B,),
        in_specs=[
            pl.BlockSpec(memory_space=pltpu.SMEM),
            pl.BlockSpec((1, NA, NB), lambda b: (b, 0, 0)),
        ],
        out_specs=[
            pl.BlockSpec((1, NA, 1), lambda b: (b, 0, 0)),
            pl.BlockSpec((1, NA, 1), lambda b: (b, 0, 0)),
        ],
        out_shape=[
            jax.ShapeDtypeStruct((B, NA, 1), jnp.int32),
            jax.ShapeDtypeStruct((B, NA, 1), jnp.float32),
        ],
    )(thr, iou)

    idx_padded = jnp.concatenate(
        [flat_idx.reshape(B * NA), jnp.zeros((PAD - B * NA,), jnp.int32)]
    ).reshape(PAD // CHUNK, CHUNK)

    mesh = plsc.VectorSubcoreMesh(core_axis_name="c", subcore_axis_name="s")
    row_out = jax.ShapeDtypeStruct((PAD, 16), jnp.float32)
    sc_dots = functools.partial(
        pl.kernel,
        out_type=[row_out] * 6,
        mesh=mesh,
        scratch_types=[
            pltpu.VMEM((1, CHUNK), jnp.int32),
            pltpu.VMEM((CHUNK, D), jnp.float32),
            pltpu.VMEM((CHUNK, D), jnp.float32),
            pltpu.VMEM((CHUNK, D), jnp.float32),
            pltpu.VMEM((CHUNK, D), jnp.float32),
            pltpu.VMEM((CHUNK, 16), jnp.float32),
            pltpu.VMEM((CHUNK, 16), jnp.float32),
            pltpu.VMEM((CHUNK, 16), jnp.float32),
            pltpu.VMEM((CHUNK, 16), jnp.float32),
            pltpu.VMEM((CHUNK, 16), jnp.float32),
            pltpu.VMEM((CHUNK, 16), jnp.float32),
            pltpu.SemaphoreType.DMA,
            pltpu.SemaphoreType.DMA,
        ],
    )(_sc_dot_body)
    da, db, nap, naz, ngp, ngz = sc_dots(
        feat_a_p.reshape(B * NA, D),
        feat_a_z.reshape(B * NA, D),
        feat_b_p.reshape(B * NB, D),
        feat_b_z.reshape(B * NB, D),
        idx_padded,
    )

    loss = da[0, 0] * 0.0 + db[0, 0] * 0.0 + nap[0, 0] * 0.0 + naz[0, 0] * 0.0 + ngp[0, 0] * 0.0 + ngz[0, 0] * 0.0
    matched_box_num = mask[:, 0, 0]
    return (loss, matched_box_num)


# speedup vs baseline: 1.0259x; 1.0259x over previous
import functools

import jax
import jax.numpy as jnp
from jax import lax
from jax.experimental import pallas as pl
from jax.experimental.pallas import tpu as pltpu
from jax.experimental.pallas import tpu_sc as plsc

B, NA, NB, D = 8, 1000, 1000, 256
N = B * NA
CHUNK = 40
NCHUNK = N // CHUNK
NW = 32
LAN = D // 16


def _tc_argmax_body(thr_ref, iou_ref, idx_ref, mask_ref):
    x = iou_ref[0]
    col = lax.broadcasted_iota(jnp.int32, (NA, NB), 1)
    mx = jnp.max(x, axis=1, keepdims=True)
    cand = jnp.where(x == mx, col, NB)
    jst = jnp.min(cand, axis=1, keepdims=True)
    b = pl.program_id(0)
    mk = (mx >= thr_ref[0]).astype(jnp.float32)
    idx_ref[...] = (jst + b * NB).T.reshape(1, 1, NA)
    mask_ref[...] = mk.T.reshape(1, 1, NA)


def _sc_dot_body(ap_hbm, az_hbm, bp_hbm, bz_hbm, idx_hbm, out_hbm,
                 idx_v, ap_v, az_v, gp_v, gz_v, o_v, sems):
    wid = lax.axis_index("s") * 2 + lax.axis_index("c")
    n_t = 7

    def copies(t, g):
        u = t % 2
        pltpu.sync_copy(idx_hbm.at[g], idx_v.at[u])
        r0 = pl.multiple_of(g * CHUNK, 8)
        pltpu.async_copy(bp_hbm.at[idx_v.at[u, 0]], gp_v.at[u], sems.at[u, 0])
        pltpu.async_copy(bz_hbm.at[idx_v.at[u, 0]], gz_v.at[u], sems.at[u, 1])
        pltpu.async_copy(ap_hbm.at[pl.ds(r0, CHUNK)], ap_v.at[u], sems.at[u, 2])
        pltpu.async_copy(az_hbm.at[pl.ds(r0, CHUNK)], az_v.at[u], sems.at[u, 3])

    def waits(t):
        u = t % 2
        pltpu.make_async_copy(bp_hbm.at[pl.ds(0, CHUNK)], gp_v.at[u], sems.at[u, 0]).wait()
        pltpu.make_async_copy(bz_hbm.at[pl.ds(0, CHUNK)], gz_v.at[u], sems.at[u, 1]).wait()
        pltpu.make_async_copy(ap_hbm.at[pl.ds(0, CHUNK)], ap_v.at[u], sems.at[u, 2]).wait()
        pltpu.make_async_copy(az_hbm.at[pl.ds(0, CHUNK)], az_v.at[u], sems.at[u, 3]).wait()

    def compute(t):
        u = t % 2

        def body(r, carry):
            da = jnp.zeros((16,), jnp.float32)
            db = jnp.zeros((16,), jnp.float32)
            nap = jnp.zeros((16,), jnp.float32)
            naz = jnp.zeros((16,), jnp.float32)
            ngp = jnp.zeros((16,), jnp.float32)
            ngz = jnp.zeros((16,), jnp.float32)
            for d in range(LAN):
                sl = pl.ds(16 * d, 16)
                ap = ap_v[u, r, sl]
                az = az_v[u, r, sl]
                gp = gp_v[u, r, sl]
                gz = gz_v[u, r, sl]
                da = da + ap * gz
                db = db + gp * az
                nap = nap + ap * ap
                naz = naz + az * az
                ngp = ngp + gp * gp
                ngz = ngz + gz * gz
            o_v[r, pl.ds(0, 16)] = da
            o_v[r, pl.ds(16, 16)] = db
            o_v[r, pl.ds(32, 16)] = nap
            o_v[r, pl.ds(48, 16)] = naz
            o_v[r, pl.ds(64, 16)] = ngp
            o_v[r, pl.ds(80, 16)] = ngz
            return carry

        lax.fori_loop(0, CHUNK, body, 0, unroll=False)

    copies(0, wid)
    for t in range(n_t):
        g = wid + NW * t

        @pl.when(g < NCHUNK)
        def _(t=t):
            waits(t)

        if t + 1 < n_t:
            g2 = wid + NW * (t + 1)

            @pl.when(g2 < NCHUNK)
            def _(t=t, g2=g2):
                copies(t + 1, g2)

        @pl.when(g < NCHUNK)
        def _(t=t, g=g):
            compute(t)
            r0 = pl.multiple_of(g * CHUNK, 8)
            pltpu.sync_copy(o_v, out_hbm.at[pl.ds(r0, CHUNK)])


def _tc_final_body(o_ref, m_ref, sa_ref, sb_ref, c_ref):
    eps = jnp.float32(1e-12)
    o = o_ref[...]
    m = m_ref[0]

    def part(k):
        return jnp.sum(o[:, 16 * k:16 * (k + 1)], axis=1, keepdims=True)

    da = part(0)
    db = part(1)
    nap = part(2)
    naz = part(3)
    ngp = part(4)
    ngz = part(5)

    def nrm(x):
        return jnp.maximum(jnp.sqrt(x), eps)

    pa = da / (nrm(nap) * nrm(ngz))
    pb = db / (nrm(ngp) * nrm(naz))
    dn = (((1,), (0,)), ((), ()))
    sa = lax.dot_general(m, pa, dn, precision=lax.Precision.HIGHEST,
                         preferred_element_type=jnp.float32)
    sb = lax.dot_general(m, pb, dn, precision=lax.Precision.HIGHEST,
                         preferred_element_type=jnp.float32)
    sa_ref[...] = jnp.broadcast_to(sa.reshape(1, 1, 1), (1, 8, 128))
    sb_ref[...] = jnp.broadcast_to(sb.reshape(1, 1, 1), (1, 8, 128))
    c_ref[...] = jnp.broadcast_to(jnp.sum(m), (1, 8, 128))


def kernel(feat_a_p, feat_a_z, feat_b_p, feat_b_z, iou, iou_threshold):
    thr = jnp.asarray(iou_threshold, jnp.float32).reshape(1)

    flat_idx, mask = pl.pallas_call(
        _tc_argmax_body,
        grid=(B,),
        in_specs=[
            pl.BlockSpec(memory_space=pltpu.SMEM),
            pl.BlockSpec((1, NA, NB), lambda b: (b, 0, 0)),
        ],
        out_specs=[
            pl.BlockSpec((1, 1, NA), lambda b: (b, 0, 0)),
            pl.BlockSpec((1, 1, NA), lambda b: (b, 0, 0)),
        ],
        out_shape=[
            jax.ShapeDtypeStruct((B, 1, NA), jnp.int32),
            jax.ShapeDtypeStruct((B, 1, NA), jnp.float32),
        ],
    )(thr, iou)

    idx3d = flat_idx.reshape(NCHUNK, 1, CHUNK)

    mesh = plsc.VectorSubcoreMesh(core_axis_name="c", subcore_axis_name="s")
    sc_dots = functools.partial(
        pl.kernel,
        out_type=jax.ShapeDtypeStruct((N, 128), jnp.float32),
        mesh=mesh,
        scratch_types=[
            pltpu.VMEM((2, 1, CHUNK), jnp.int32),
            pltpu.VMEM((2, CHUNK, D), jnp.float32),
            pltpu.VMEM((2, CHUNK, D), jnp.float32),
            pltpu.VMEM((2, CHUNK, D), jnp.float32),
            pltpu.VMEM((2, CHUNK, D), jnp.float32),
            pltpu.VMEM((CHUNK, 128), jnp.float32),
            pltpu.SemaphoreType.DMA((2, 4)),
        ],
    )(_sc_dot_body)
    packed = sc_dots(
        feat_a_p.reshape(N, D),
        feat_a_z.reshape(N, D),
        feat_b_p.reshape(B * NB, D),
        feat_b_z.reshape(B * NB, D),
        idx3d,
    )

    sa, sb, cnt = pl.pallas_call(
        _tc_final_body,
        grid=(B,),
        in_specs=[
            pl.BlockSpec((NA, 128), lambda b: (b, 0)),
            pl.BlockSpec((1, 1, NA), lambda b: (b, 0, 0)),
        ],
        out_specs=[pl.BlockSpec((1, 8, 128), lambda b: (b, 0, 0))] * 3,
        out_shape=[jax.ShapeDtypeStruct((B, 8, 128), jnp.float32)] * 3,
    )(packed, mask)

    matched_box_num = cnt[:, 0, 0]
    denom = jnp.maximum(jnp.sum(matched_box_num), 1.0)
    loss = -(jnp.sum(sa[:, 0, 0]) + jnp.sum(sb[:, 0, 0])) / (2.0 * denom)
    return (loss, matched_box_num)


# speedup vs baseline: 1.1149x; 1.0867x over previous
import functools

import jax
import jax.numpy as jnp
from jax import lax
from jax.experimental import pallas as pl
from jax.experimental.pallas import tpu as pltpu
from jax.experimental.pallas import tpu_sc as plsc

B, NA, NB, D = 8, 1000, 1000, 256
N = B * NA
CHUNK = 40
NCHUNK = N // CHUNK
NW = 32
LAN = D // 16


def _tc_argmax_body(thr_ref, iou_ref, idx_ref, mask_ref):
    x = iou_ref[0]
    col = lax.broadcasted_iota(jnp.int32, (NA, NB), 1)
    mx = jnp.max(x, axis=1, keepdims=True)
    cand = jnp.where(x == mx, col, NB)
    jst = jnp.min(cand, axis=1, keepdims=True)
    b = pl.program_id(0)
    mk = (mx >= thr_ref[0]).astype(jnp.float32)
    idx_ref[...] = (jst + b * NB).T.reshape(1, 1, NA)
    mask_ref[...] = mk.T.reshape(1, 1, NA)


def _sc_dot_body(ap_hbm, az_hbm, bp_hbm, bz_hbm, idx_hbm, out_hbm,
                 idx_v, ap_v, az_v, gp_v, gz_v, o_v, sems):
    wid = lax.axis_index("s") * 2 + lax.axis_index("c")
    n_t = 7

    def copies(t, g):
        u = t % 2
        pltpu.sync_copy(idx_hbm.at[g], idx_v.at[u])
        r0 = pl.multiple_of(g * CHUNK, 8)
        pltpu.async_copy(bp_hbm.at[idx_v.at[u, 0]], gp_v.at[u], sems.at[u, 0])
        pltpu.async_copy(bz_hbm.at[idx_v.at[u, 0]], gz_v.at[u], sems.at[u, 1])
        pltpu.async_copy(ap_hbm.at[pl.ds(r0, CHUNK)], ap_v.at[u], sems.at[u, 2])
        pltpu.async_copy(az_hbm.at[pl.ds(r0, CHUNK)], az_v.at[u], sems.at[u, 3])

    def waits(t):
        u = t % 2
        pltpu.make_async_copy(bp_hbm.at[pl.ds(0, CHUNK)], gp_v.at[u], sems.at[u, 0]).wait()
        pltpu.make_async_copy(bz_hbm.at[pl.ds(0, CHUNK)], gz_v.at[u], sems.at[u, 1]).wait()
        pltpu.make_async_copy(ap_hbm.at[pl.ds(0, CHUNK)], ap_v.at[u], sems.at[u, 2]).wait()
        pltpu.make_async_copy(az_hbm.at[pl.ds(0, CHUNK)], az_v.at[u], sems.at[u, 3]).wait()

    def compute(t):
        u = t % 2

        def body(r, carry):
            da = jnp.zeros((16,), jnp.float32)
            db = jnp.zeros((16,), jnp.float32)
            nap = jnp.zeros((16,), jnp.float32)
            naz = jnp.zeros((16,), jnp.float32)
            ngp = jnp.zeros((16,), jnp.float32)
            ngz = jnp.zeros((16,), jnp.float32)
            for d in range(LAN):
                sl = pl.ds(16 * d, 16)
                ap = ap_v[u, r, sl]
                az = az_v[u, r, sl]
                gp = gp_v[u, r, sl]
                gz = gz_v[u, r, sl]
                da = da + ap * gz
                db = db + gp * az
                nap = nap + ap * ap
                naz = naz + az * az
                ngp = ngp + gp * gp
                ngz = ngz + gz * gz
            o_v[r, pl.ds(0, 16)] = da
            o_v[r, pl.ds(16, 16)] = db
            o_v[r, pl.ds(32, 16)] = nap
            o_v[r, pl.ds(48, 16)] = naz
            o_v[r, pl.ds(64, 16)] = ngp
            o_v[r, pl.ds(80, 16)] = ngz
            return carry

        lax.fori_loop(0, CHUNK, body, 0, unroll=False)

    copies(0, wid)
    for t in range(n_t):
        g = wid + NW * t

        @pl.when(g < NCHUNK)
        def _(t=t):
            waits(t)

        if t + 1 < n_t:
            g2 = wid + NW * (t + 1)

            @pl.when(g2 < NCHUNK)
            def _(t=t, g2=g2):
                copies(t + 1, g2)

        @pl.when(g < NCHUNK)
        def _(t=t, g=g):
            compute(t)
            r0 = pl.multiple_of(g * CHUNK, 8)
            pltpu.sync_copy(o_v, out_hbm.at[pl.ds(r0, CHUNK)])


def _tc_final_body(o_ref, m_ref, loss_ref, c_ref):
    eps = jnp.float32(1e-12)
    o = o_ref[...]
    m3 = m_ref[...]

    def part(k):
        return jnp.sum(o[:, 16 * k:16 * (k + 1)], axis=1, keepdims=True)

    da = part(0)
    db = part(1)
    nap = part(2)
    naz = part(3)
    ngp = part(4)
    ngz = part(5)

    def nrm(x):
        return jnp.maximum(jnp.sqrt(x), eps)

    pa = (da / (nrm(nap) * nrm(ngz))).reshape(B, NA, 1)
    pb = (db / (nrm(ngp) * nrm(naz))).reshape(B, NA, 1)
    dn = (((2,), (1,)), ((0,), (0,)))
    sa = lax.dot_general(m3, pa, dn, precision=lax.Precision.HIGHEST,
                         preferred_element_type=jnp.float32)
    sb = lax.dot_general(m3, pb, dn, precision=lax.Precision.HIGHEST,
                         preferred_element_type=jnp.float32)
    cnt = jnp.sum(m3, axis=2)
    denom = jnp.maximum(jnp.sum(cnt), 1.0)
    loss = -(jnp.sum(sa) + jnp.sum(sb)) / (2.0 * denom)
    loss_ref[...] = jnp.broadcast_to(loss, (8, 128))
    c_ref[...] = jnp.broadcast_to(cnt, (B, 128))


def kernel(feat_a_p, feat_a_z, feat_b_p, feat_b_z, iou, iou_threshold):
    thr = jnp.asarray(iou_threshold, jnp.float32).reshape(1)

    flat_idx, mask = pl.pallas_call(
        _tc_argmax_body,
        grid=(B,),
        in_specs=[
            pl.BlockSpec(memory_space=pltpu.SMEM),
            pl.BlockSpec((1, NA, NB), lambda b: (b, 0, 0)),
        ],
        out_specs=[
            pl.BlockSpec((1, 1, NA), lambda b: (b, 0, 0)),
            pl.BlockSpec((1, 1, NA), lambda b: (b, 0, 0)),
        ],
        out_shape=[
            jax.ShapeDtypeStruct((B, 1, NA), jnp.int32),
            jax.ShapeDtypeStruct((B, 1, NA), jnp.float32),
        ],
    )(thr, iou)

    idx3d = flat_idx.reshape(NCHUNK, 1, CHUNK)

    mesh = plsc.VectorSubcoreMesh(core_axis_name="c", subcore_axis_name="s")
    sc_dots = functools.partial(
        pl.kernel,
        out_type=jax.ShapeDtypeStruct((N, 128), jnp.float32),
        mesh=mesh,
        scratch_types=[
            pltpu.VMEM((2, 1, CHUNK), jnp.int32),
            pltpu.VMEM((2, CHUNK, D), jnp.float32),
            pltpu.VMEM((2, CHUNK, D), jnp.float32),
            pltpu.VMEM((2, CHUNK, D), jnp.float32),
            pltpu.VMEM((2, CHUNK, D), jnp.float32),
            pltpu.VMEM((CHUNK, 128), jnp.float32),
            pltpu.SemaphoreType.DMA((2, 4)),
        ],
    )(_sc_dot_body)
    packed = sc_dots(
        feat_a_p.reshape(N, D),
        feat_a_z.reshape(N, D),
        feat_b_p.reshape(B * NB, D),
        feat_b_z.reshape(B * NB, D),
        idx3d,
    )

    loss_o, cnt_o = pl.pallas_call(
        _tc_final_body,
        grid=(1,),
        in_specs=[
            pl.BlockSpec((N, 128), lambda i: (0, 0)),
            pl.BlockSpec((B, 1, NA), lambda i: (0, 0, 0)),
        ],
        out_specs=[pl.BlockSpec((8, 128), lambda i: (0, 0))] * 2,
        out_shape=[jax.ShapeDtypeStruct((8, 128), jnp.float32)] * 2,
    )(packed, mask)

    return (loss_o[0, 0], cnt_o[:, 0])


# speedup vs baseline: 1.1333x; 1.0166x over previous
import functools

import jax
import jax.numpy as jnp
from jax import lax
from jax.experimental import pallas as pl
from jax.experimental.pallas import tpu as pltpu
from jax.experimental.pallas import tpu_sc as plsc

B, NA, NB, D = 8, 1000, 1000, 256
N = B * NA
CHUNK = 40
NCHUNK = N // CHUNK
NW = 32
LAN = D // 16


def _tc_argmax_body(thr_ref, iou_ref, idx_ref, mask_ref):
    x = iou_ref[0]
    col = lax.broadcasted_iota(jnp.int32, (NA, NB), 1)
    mx = jnp.max(x, axis=1, keepdims=True)
    cand = jnp.where(x == mx, col, NB)
    jst = jnp.min(cand, axis=1, keepdims=True)
    b = pl.program_id(0)
    mk = (mx >= thr_ref[0]).astype(jnp.float32)
    idx_ref[...] = (jst + b * NB).T.reshape(1, 1, NA)
    mask_ref[...] = mk.T.reshape(1, 1, NA)


def _sc_dot_body(ap_hbm, az_hbm, bp_hbm, bz_hbm, idx_hbm, out_hbm,
                 idx_v, ap_v, az_v, gp_v, gz_v, o_v, sems):
    wid = lax.axis_index("s") * 2 + lax.axis_index("c")
    n_t = 7

    def copies(t, g):
        u = t % 2
        pltpu.sync_copy(idx_hbm.at[g], idx_v.at[u])
        r0 = pl.multiple_of(g * CHUNK, 8)
        pltpu.async_copy(bp_hbm.at[idx_v.at[u, 0]], gp_v.at[u], sems.at[u, 0])
        pltpu.async_copy(bz_hbm.at[idx_v.at[u, 0]], gz_v.at[u], sems.at[u, 1])
        pltpu.async_copy(ap_hbm.at[pl.ds(r0, CHUNK)], ap_v.at[u], sems.at[u, 2])
        pltpu.async_copy(az_hbm.at[pl.ds(r0, CHUNK)], az_v.at[u], sems.at[u, 3])

    def waits(t):
        u = t % 2
        pltpu.make_async_copy(bp_hbm.at[pl.ds(0, CHUNK)], gp_v.at[u], sems.at[u, 0]).wait()
        pltpu.make_async_copy(bz_hbm.at[pl.ds(0, CHUNK)], gz_v.at[u], sems.at[u, 1]).wait()
        pltpu.make_async_copy(ap_hbm.at[pl.ds(0, CHUNK)], ap_v.at[u], sems.at[u, 2]).wait()
        pltpu.make_async_copy(az_hbm.at[pl.ds(0, CHUNK)], az_v.at[u], sems.at[u, 3]).wait()

    def compute(t):
        u = t % 2

        def body(r, carry):
            da = jnp.zeros((16,), jnp.float32)
            db = jnp.zeros((16,), jnp.float32)
            nap = jnp.zeros((16,), jnp.float32)
            naz = jnp.zeros((16,), jnp.float32)
            ngp = jnp.zeros((16,), jnp.float32)
            ngz = jnp.zeros((16,), jnp.float32)
            for d in range(LAN):
                sl = pl.ds(16 * d, 16)
                ap = ap_v[u, r, sl]
                az = az_v[u, r, sl]
                gp = gp_v[u, r, sl]
                gz = gz_v[u, r, sl]
                da = da + ap * gz
                db = db + gp * az
                nap = nap + ap * ap
                naz = naz + az * az
                ngp = ngp + gp * gp
                ngz = ngz + gz * gz
            o_v[r, pl.ds(0, 16)] = da
            o_v[r, pl.ds(16, 16)] = db
            o_v[r, pl.ds(32, 16)] = nap
            o_v[r, pl.ds(48, 16)] = naz
            o_v[r, pl.ds(64, 16)] = ngp
            o_v[r, pl.ds(80, 16)] = ngz
            return carry

        lax.fori_loop(0, CHUNK, body, 0, unroll=2)

    copies(0, wid)
    for t in range(n_t):
        g = wid + NW * t

        @pl.when(g < NCHUNK)
        def _(t=t):
            waits(t)

        if t + 1 < n_t:
            g2 = wid + NW * (t + 1)

            @pl.when(g2 < NCHUNK)
            def _(t=t, g2=g2):
                copies(t + 1, g2)

        @pl.when(g < NCHUNK)
        def _(t=t, g=g):
            compute(t)
            r0 = pl.multiple_of(g * CHUNK, 8)
            pltpu.sync_copy(o_v, out_hbm.at[pl.ds(r0, CHUNK)])


def _tc_final_body(o_ref, m_ref, loss_ref, c_ref):
    eps = jnp.float32(1e-12)
    o = o_ref[...]
    m3 = m_ref[...]

    def part(k):
        return jnp.sum(o[:, 16 * k:16 * (k + 1)], axis=1, keepdims=True)

    da = part(0)
    db = part(1)
    nap = part(2)
    naz = part(3)
    ngp = part(4)
    ngz = part(5)

    def nrm(x):
        return jnp.maximum(jnp.sqrt(x), eps)

    pa = (da / (nrm(nap) * nrm(ngz))).reshape(B, NA, 1)
    pb = (db / (nrm(ngp) * nrm(naz))).reshape(B, NA, 1)
    dn = (((2,), (1,)), ((0,), (0,)))
    sa = lax.dot_general(m3, pa, dn, precision=lax.Precision.HIGHEST,
                         preferred_element_type=jnp.float32)
    sb = lax.dot_general(m3, pb, dn, precision=lax.Precision.HIGHEST,
                         preferred_element_type=jnp.float32)
    cnt = jnp.sum(m3, axis=2)
    denom = jnp.maximum(jnp.sum(cnt), 1.0)
    loss = -(jnp.sum(sa) + jnp.sum(sb)) / (2.0 * denom)
    loss_ref[...] = jnp.broadcast_to(loss, (8, 128))
    c_ref[...] = jnp.broadcast_to(cnt, (B, 128))


def kernel(feat_a_p, feat_a_z, feat_b_p, feat_b_z, iou, iou_threshold):
    thr = jnp.asarray(iou_threshold, jnp.float32).reshape(1)

    flat_idx, mask = pl.pallas_call(
        _tc_argmax_body,
        grid=(B,),
        in_specs=[
            pl.BlockSpec(memory_space=pltpu.SMEM),
            pl.BlockSpec((1, NA, NB), lambda b: (b, 0, 0)),
        ],
        out_specs=[
            pl.BlockSpec((1, 1, NA), lambda b: (b, 0, 0)),
            pl.BlockSpec((1, 1, NA), lambda b: (b, 0, 0)),
        ],
        out_shape=[
            jax.ShapeDtypeStruct((B, 1, NA), jnp.int32),
            jax.ShapeDtypeStruct((B, 1, NA), jnp.float32),
        ],
    )(thr, iou)

    idx3d = flat_idx.reshape(NCHUNK, 1, CHUNK)

    mesh = plsc.VectorSubcoreMesh(core_axis_name="c", subcore_axis_name="s")
    sc_dots = functools.partial(
        pl.kernel,
        out_type=jax.ShapeDtypeStruct((N, 128), jnp.float32),
        mesh=mesh,
        scratch_types=[
            pltpu.VMEM((2, 1, CHUNK), jnp.int32),
            pltpu.VMEM((2, CHUNK, D), jnp.float32),
            pltpu.VMEM((2, CHUNK, D), jnp.float32),
            pltpu.VMEM((2, CHUNK, D), jnp.float32),
            pltpu.VMEM((2, CHUNK, D), jnp.float32),
            pltpu.VMEM((CHUNK, 128), jnp.float32),
            pltpu.SemaphoreType.DMA((2, 4)),
        ],
    )(_sc_dot_body)
    packed = sc_dots(
        feat_a_p.reshape(N, D),
        feat_a_z.reshape(N, D),
        feat_b_p.reshape(B * NB, D),
        feat_b_z.reshape(B * NB, D),
        idx3d,
    )

    loss_o, cnt_o = pl.pallas_call(
        _tc_final_body,
        grid=(1,),
        in_specs=[
            pl.BlockSpec((N, 128), lambda i: (0, 0)),
            pl.BlockSpec((B, 1, NA), lambda i: (0, 0, 0)),
        ],
        out_specs=[pl.BlockSpec((8, 128), lambda i: (0, 0))] * 2,
        out_shape=[jax.ShapeDtypeStruct((8, 128), jnp.float32)] * 2,
    )(packed, mask)

    return (loss_o[0, 0], cnt_o[:, 0])


# speedup vs baseline: 1.2616x; 1.1131x over previous
import functools

import jax
import jax.numpy as jnp
from jax import lax
from jax.experimental import pallas as pl
from jax.experimental.pallas import tpu as pltpu
from jax.experimental.pallas import tpu_sc as plsc

B, NA, NB, D = 8, 1000, 1000, 256
N = B * NA
CHUNK = 40
NCHUNK = N // CHUNK
NW = 32
LAN = D // 16


def _tc_argmax_body(thr_ref, iou_ref, idx_ref, mask_ref):
    x = iou_ref[0]
    col = lax.broadcasted_iota(jnp.int32, (NA, NB), 1)
    mx = jnp.max(x, axis=1, keepdims=True)
    cand = jnp.where(x == mx, col, NB)
    jst = jnp.min(cand, axis=1, keepdims=True)
    b = pl.program_id(0)
    mk = (mx >= thr_ref[0]).astype(jnp.float32)
    idx_ref[...] = (jst + b * NB).T.reshape(1, 1, NA)
    mask_ref[...] = mk.T.reshape(1, 1, NA)


def _sc_dot_body(ap_hbm, az_hbm, bp_hbm, bz_hbm, idx_hbm, out_hbm,
                 idx_v, ap_v, az_v, gp_v, gz_v, o_v, sems):
    wid = lax.axis_index("s") * 2 + lax.axis_index("c")
    n_t = 7

    def copies(t, g):
        u = t % 2
        pltpu.sync_copy(idx_hbm.at[g], idx_v.at[u])
        r0 = pl.multiple_of(g * CHUNK, 8)
        pltpu.async_copy(bp_hbm.at[idx_v.at[u, 0]], gp_v.at[u], sems.at[u, 0])
        pltpu.async_copy(bz_hbm.at[idx_v.at[u, 0]], gz_v.at[u], sems.at[u, 1])
        pltpu.async_copy(ap_hbm.at[pl.ds(r0, CHUNK)], ap_v.at[u], sems.at[u, 2])
        pltpu.async_copy(az_hbm.at[pl.ds(r0, CHUNK)], az_v.at[u], sems.at[u, 3])

    def waits(t):
        u = t % 2
        pltpu.make_async_copy(bp_hbm.at[pl.ds(0, CHUNK)], gp_v.at[u], sems.at[u, 0]).wait()
        pltpu.make_async_copy(bz_hbm.at[pl.ds(0, CHUNK)], gz_v.at[u], sems.at[u, 1]).wait()
        pltpu.make_async_copy(ap_hbm.at[pl.ds(0, CHUNK)], ap_v.at[u], sems.at[u, 2]).wait()
        pltpu.make_async_copy(az_hbm.at[pl.ds(0, CHUNK)], az_v.at[u], sems.at[u, 3]).wait()

    def compute(t):
        u = t % 2

        def body(r, carry):
            da = jnp.zeros((16,), jnp.float32)
            db = jnp.zeros((16,), jnp.float32)
            nap = jnp.zeros((16,), jnp.float32)
            naz = jnp.zeros((16,), jnp.float32)
            ngp = jnp.zeros((16,), jnp.float32)
            ngz = jnp.zeros((16,), jnp.float32)
            for d in range(LAN):
                sl = pl.ds(16 * d, 16)
                ap = ap_v[u, r, sl]
                az = az_v[u, r, sl]
                gp = gp_v[u, r, sl]
                gz = gz_v[u, r, sl]
                da = da + ap * gz
                db = db + gp * az
                nap = nap + ap * ap
                naz = naz + az * az
                ngp = ngp + gp * gp
                ngz = ngz + gz * gz
            o_v[r, pl.ds(0, 16)] = da
            o_v[r, pl.ds(16, 16)] = db
            o_v[r, pl.ds(32, 16)] = nap
            o_v[r, pl.ds(48, 16)] = naz
            o_v[r, pl.ds(64, 16)] = ngp
            o_v[r, pl.ds(80, 16)] = ngz
            return carry

        lax.fori_loop(0, CHUNK, body, 0, unroll=2)

    copies(0, wid)
    for t in range(n_t):
        g = wid + NW * t

        @pl.when(g < NCHUNK)
        def _(t=t):
            waits(t)

        if t + 1 < n_t:
            g2 = wid + NW * (t + 1)

            @pl.when(g2 < NCHUNK)
            def _(t=t, g2=g2):
                copies(t + 1, g2)

        @pl.when(g < NCHUNK)
        def _(t=t, g=g):
            compute(t)
            r0 = pl.multiple_of(g * CHUNK, 8)
            pltpu.sync_copy(o_v, out_hbm.at[pl.ds(r0, CHUNK)])


def _tc_final_body(o_ref, m_ref, loss_ref, c_ref, acc_ref):
    eps = jnp.float32(1e-12)
    b = pl.program_id(0)
    o = o_ref[...]
    m = m_ref[0]

    ei = (lax.broadcasted_iota(jnp.int32, (128, 8), 0) // 16 ==
          lax.broadcasted_iota(jnp.int32, (128, 8), 1)).astype(jnp.float32)
    r = lax.dot_general(ei, o, (((0,), (1,)), ((), ())),
                        precision=lax.Precision.HIGHEST,
                        preferred_element_type=jnp.float32)
    da, db, nap, naz, ngp, ngz = (r[k:k + 1, :] for k in range(6))

    def nrm(x):
        return jnp.maximum(jnp.sqrt(x), eps)

    pa = da / (nrm(nap) * nrm(ngz))
    pb = db / (nrm(ngp) * nrm(naz))
    sa_b = jnp.sum(m * pa)
    sb_b = jnp.sum(m * pb)
    cnt_b = jnp.sum(m)

    @pl.when(b == 0)
    def _():
        acc_ref[0] = 0.0
        acc_ref[1] = 0.0

    acc_ref[0] = acc_ref[0] + sa_b + sb_b
    acc_ref[1] = acc_ref[1] + cnt_b
    c_ref[b, 0] = cnt_b

    @pl.when(b == B - 1)
    def _():
        denom = jnp.maximum(acc_ref[1], 1.0)
        loss_ref[...] = jnp.broadcast_to(-acc_ref[0] / (2.0 * denom), (8, 128))


def kernel(feat_a_p, feat_a_z, feat_b_p, feat_b_z, iou, iou_threshold):
    thr = jnp.asarray(iou_threshold, jnp.float32).reshape(1)

    flat_idx, mask = pl.pallas_call(
        _tc_argmax_body,
        grid=(B,),
        in_specs=[
            pl.BlockSpec(memory_space=pltpu.SMEM),
            pl.BlockSpec((1, NA, NB), lambda b: (b, 0, 0)),
        ],
        out_specs=[
            pl.BlockSpec((1, 1, NA), lambda b: (b, 0, 0)),
            pl.BlockSpec((1, 1, NA), lambda b: (b, 0, 0)),
        ],
        out_shape=[
            jax.ShapeDtypeStruct((B, 1, NA), jnp.int32),
            jax.ShapeDtypeStruct((B, 1, NA), jnp.float32),
        ],
    )(thr, iou)

    idx3d = flat_idx.reshape(NCHUNK, 1, CHUNK)

    mesh = plsc.VectorSubcoreMesh(core_axis_name="c", subcore_axis_name="s")
    sc_dots = functools.partial(
        pl.kernel,
        out_type=jax.ShapeDtypeStruct((N, 128), jnp.float32),
        mesh=mesh,
        scratch_types=[
            pltpu.VMEM((2, 1, CHUNK), jnp.int32),
            pltpu.VMEM((2, CHUNK, D), jnp.float32),
            pltpu.VMEM((2, CHUNK, D), jnp.float32),
            pltpu.VMEM((2, CHUNK, D), jnp.float32),
            pltpu.VMEM((2, CHUNK, D), jnp.float32),
            pltpu.VMEM((CHUNK, 128), jnp.float32),
            pltpu.SemaphoreType.DMA((2, 4)),
        ],
    )(_sc_dot_body)
    packed = sc_dots(
        feat_a_p.reshape(N, D),
        feat_a_z.reshape(N, D),
        feat_b_p.reshape(B * NB, D),
        feat_b_z.reshape(B * NB, D),
        idx3d,
    )

    loss_o, cnt_o = pl.pallas_call(
        _tc_final_body,
        grid=(B,),
        in_specs=[
            pl.BlockSpec((NA, 128), lambda b: (b, 0)),
            pl.BlockSpec((1, 1, NA), lambda b: (b, 0, 0)),
        ],
        out_specs=[
            pl.BlockSpec((8, 128), lambda b: (0, 0)),
            pl.BlockSpec(memory_space=pltpu.SMEM),
        ],
        out_shape=[
            jax.ShapeDtypeStruct((8, 128), jnp.float32),
            jax.ShapeDtypeStruct((B, 1), jnp.float32),
        ],
        scratch_shapes=[pltpu.SMEM((2,), jnp.float32)],
    )(packed, mask)

    return (loss_o[0, 0], cnt_o[:, 0])


# speedup vs baseline: 1.3800x; 1.0939x over previous
import functools

import jax
import jax.numpy as jnp
from jax import lax
from jax.experimental import pallas as pl
from jax.experimental.pallas import tpu as pltpu
from jax.experimental.pallas import tpu_sc as plsc

B, NA, NB, D = 8, 1000, 1000, 256
N = B * NA
CHUNK = 40
NCHUNK = N // CHUNK
NW = 32
LAN = D // 16
EPS = 1e-12


def _tc_argmax_body(thr_ref, iou_ref, idx_ref, mask_ref):
    x = iou_ref[0]
    col = lax.broadcasted_iota(jnp.int32, (NA, NB), 1)
    mx = jnp.max(x, axis=1, keepdims=True)
    cand = jnp.where(x == mx, col, NB)
    jst = jnp.min(cand, axis=1, keepdims=True)
    b = pl.program_id(0)
    mk = (mx >= thr_ref[0]).astype(jnp.float32)
    idx_ref[...] = (jst + b * NB).T.reshape(1, 1, NA)
    mask_ref[...] = mk.T.reshape(1, 1, NA)


def _sc_dot_body(az_hbm, bp_hbm, idx_hbm, out_hbm,
                 idx_v, az_v, gp_v, o_v, sems):
    wid = lax.axis_index("s") * 2 + lax.axis_index("c")
    n_t = 7

    def copies(t, g):
        u = t % 2
        pltpu.sync_copy(idx_hbm.at[g], idx_v.at[u])
        r0 = pl.multiple_of(g * CHUNK, 8)
        pltpu.async_copy(bp_hbm.at[idx_v.at[u, 0]], gp_v.at[u], sems.at[u, 0])
        pltpu.async_copy(az_hbm.at[pl.ds(r0, CHUNK)], az_v.at[u], sems.at[u, 1])

    def waits(t):
        u = t % 2
        pltpu.make_async_copy(bp_hbm.at[pl.ds(0, CHUNK)], gp_v.at[u], sems.at[u, 0]).wait()
        pltpu.make_async_copy(az_hbm.at[pl.ds(0, CHUNK)], az_v.at[u], sems.at[u, 1]).wait()

    def compute(t):
        u = t % 2

        def body(r, carry):
            db = jnp.zeros((16,), jnp.float32)
            ngp = jnp.zeros((16,), jnp.float32)
            naz = jnp.zeros((16,), jnp.float32)
            for d in range(LAN):
                sl = pl.ds(16 * d, 16)
                az = az_v[u, r, sl]
                gp = gp_v[u, r, sl]
                db = db + gp * az
                ngp = ngp + gp * gp
                naz = naz + az * az
            o_v[r, pl.ds(0, 16)] = db
            o_v[r, pl.ds(16, 16)] = ngp
            o_v[r, pl.ds(32, 16)] = naz
            return carry

        lax.fori_loop(0, CHUNK, body, 0, unroll=2)

    copies(0, wid)
    for t in range(n_t):
        g = wid + NW * t

        @pl.when(g < NCHUNK)
        def _(t=t):
            waits(t)

        if t + 1 < n_t:
            g2 = wid + NW * (t + 1)

            @pl.when(g2 < NCHUNK)
            def _(t=t, g2=g2):
                copies(t + 1, g2)

        @pl.when(g < NCHUNK)
        def _(t=t, g=g):
            compute(t)
            r0 = pl.multiple_of(g * CHUNK, 8)
            pltpu.sync_copy(o_v, out_hbm.at[pl.ds(r0, CHUNK)])


def _tc_onehot_body(idx_ref, m_ref, ap_ref, bz_ref, sa_ref, acc_ref):
    b = pl.program_id(0)
    jl = idx_ref[0] - b * NB
    m = m_ref[0]
    ap = ap_ref[0]
    bz = bz_ref[0]

    def nrm_rows(x):
        n = jnp.sqrt(jnp.sum(x * x, axis=1, keepdims=True))
        return x / jnp.maximum(n, EPS)

    nap = nrm_rows(ap)
    nbz = nrm_rows(bz)
    rowj = lax.broadcasted_iota(jnp.int32, (NB, NA), 0)
    ot = ((rowj == jnp.broadcast_to(jl, (NB, NA)))
          & (jnp.broadcast_to(m, (NB, NA)) >= 0.5)).astype(jnp.bfloat16)
    hi = nap.astype(jnp.bfloat16)
    lo = (nap - hi.astype(jnp.float32)).astype(jnp.bfloat16)
    dn = (((1,), (0,)), ((), ()))
    w = (lax.dot_general(ot, hi, dn, preferred_element_type=jnp.float32)
         + lax.dot_general(ot, lo, dn, preferred_element_type=jnp.float32))
    sa_b = jnp.sum(w * nbz)

    @pl.when(b == 0)
    def _():
        acc_ref[0] = 0.0

    acc_ref[0] = acc_ref[0] + sa_b

    @pl.when(b == B - 1)
    def _():
        sa_ref[0, 0] = acc_ref[0]


def _tc_final_body(sa_ref, o_ref, m_ref, loss_ref, c_ref, acc_ref):
    b = pl.program_id(0)
    o = o_ref[...]
    m = m_ref[0]

    ei = (lax.broadcasted_iota(jnp.int32, (64, 8), 0) // 16 ==
          lax.broadcasted_iota(jnp.int32, (64, 8), 1)).astype(jnp.float32)
    r = lax.dot_general(ei, o, (((0,), (1,)), ((), ())),
                        precision=lax.Precision.HIGHEST,
                        preferred_element_type=jnp.float32)
    db = r[0:1, :]
    ngp = r[1:2, :]
    naz = r[2:3, :]

    def nrm(x):
        return jnp.maximum(jnp.sqrt(x), EPS)

    pb = db / (nrm(ngp) * nrm(naz))
    sb_b = jnp.sum(m * pb)
    cnt_b = jnp.sum(m)

    @pl.when(b == 0)
    def _():
        acc_ref[0] = 0.0
        acc_ref[1] = 0.0

    acc_ref[0] = acc_ref[0] + sb_b
    acc_ref[1] = acc_ref[1] + cnt_b
    c_ref[b, 0] = cnt_b

    @pl.when(b == B - 1)
    def _():
        denom = jnp.maximum(acc_ref[1], 1.0)
        total = acc_ref[0] + sa_ref[0, 0]
        loss_ref[...] = jnp.broadcast_to(-total / (2.0 * denom), (8, 128))


def kernel(feat_a_p, feat_a_z, feat_b_p, feat_b_z, iou, iou_threshold):
    thr = jnp.asarray(iou_threshold, jnp.float32).reshape(1)

    flat_idx, mask = pl.pallas_call(
        _tc_argmax_body,
        grid=(B,),
        in_specs=[
            pl.BlockSpec(memory_space=pltpu.SMEM),
            pl.BlockSpec((1, NA, NB), lambda b: (b, 0, 0)),
        ],
        out_specs=[
            pl.BlockSpec((1, 1, NA), lambda b: (b, 0, 0)),
            pl.BlockSpec((1, 1, NA), lambda b: (b, 0, 0)),
        ],
        out_shape=[
            jax.ShapeDtypeStruct((B, 1, NA), jnp.int32),
            jax.ShapeDtypeStruct((B, 1, NA), jnp.float32),
        ],
    )(thr, iou)

    idx3d = flat_idx.reshape(NCHUNK, 1, CHUNK)

    mesh = plsc.VectorSubcoreMesh(core_axis_name="c", subcore_axis_name="s")
    sc_dots = functools.partial(
        pl.kernel,
        out_type=jax.ShapeDtypeStruct((N, 64), jnp.float32),
        mesh=mesh,
        scratch_types=[
            pltpu.VMEM((2, 1, CHUNK), jnp.int32),
            pltpu.VMEM((2, CHUNK, D), jnp.float32),
            pltpu.VMEM((2, CHUNK, D), jnp.float32),
            pltpu.VMEM((CHUNK, 64), jnp.float32),
            pltpu.SemaphoreType.DMA((2, 2)),
        ],
    )(_sc_dot_body)
    packed = sc_dots(
        feat_a_z.reshape(N, D),
        feat_b_p.reshape(B * NB, D),
        idx3d,
    )

    sa_o = pl.pallas_call(
        _tc_onehot_body,
        grid=(B,),
        in_specs=[
            pl.BlockSpec((1, 1, NA), lambda b: (b, 0, 0)),
            pl.BlockSpec((1, 1, NA), lambda b: (b, 0, 0)),
            pl.BlockSpec((1, NA, D), lambda b: (b, 0, 0)),
            pl.BlockSpec((1, NB, D), lambda b: (b, 0, 0)),
        ],
        out_specs=pl.BlockSpec(memory_space=pltpu.SMEM),
        out_shape=jax.ShapeDtypeStruct((1, 1), jnp.float32),
        scratch_shapes=[pltpu.SMEM((1,), jnp.float32)],
    )(flat_idx, mask, feat_a_p, feat_b_z)

    loss_o, cnt_o = pl.pallas_call(
        _tc_final_body,
        grid=(B,),
        in_specs=[
            pl.BlockSpec(memory_space=pltpu.SMEM),
            pl.BlockSpec((NA, 64), lambda b: (b, 0)),
            pl.BlockSpec((1, 1, NA), lambda b: (b, 0, 0)),
        ],
        out_specs=[
            pl.BlockSpec((8, 128), lambda b: (0, 0)),
            pl.BlockSpec(memory_space=pltpu.SMEM),
        ],
        out_shape=[
            jax.ShapeDtypeStruct((8, 128), jnp.float32),
            jax.ShapeDtypeStruct((B, 1), jnp.float32),
        ],
        scratch_shapes=[pltpu.SMEM((2,), jnp.float32)],
    )(sa_o, packed, mask)

    return (loss_o[0, 0], cnt_o[:, 0])


# speedup vs baseline: 1.4533x; 1.0531x over previous
import functools

import jax
import jax.numpy as jnp
from jax import lax
from jax.experimental import pallas as pl
from jax.experimental.pallas import tpu as pltpu
from jax.experimental.pallas import tpu_sc as plsc

B, NA, NB, D = 8, 1000, 1000, 256
N = B * NA
CHUNK = 40
NCHUNK = N // CHUNK
NW = 32
LAN = D // 16
EPS = 1e-12


def _tc_argmax_body(thr_ref, iou_ref, idx_ref, mask_ref):
    x = iou_ref[0]
    col = lax.broadcasted_iota(jnp.int32, (NA, NB), 1)
    mx = jnp.max(x, axis=1, keepdims=True)
    cand = jnp.where(x == mx, col, NB)
    jst = jnp.min(cand, axis=1, keepdims=True)
    b = pl.program_id(0)
    mk = (mx >= thr_ref[0]).astype(jnp.float32)
    idx_ref[...] = (jst + b * NB).T.reshape(1, 1, NA)
    mask_ref[...] = mk.T.reshape(1, 1, NA)


def _sc_dot_body(az_hbm, bp_hbm, idx_hbm, out_hbm,
                 idx_v, az_v, gp_v, o_v, sems):
    wid = lax.axis_index("s") * 2 + lax.axis_index("c")
    n_t = 7

    def copies(t, g):
        u = t % 2
        pltpu.sync_copy(idx_hbm.at[g], idx_v.at[u])
        r0 = pl.multiple_of(g * CHUNK, 8)
        pltpu.async_copy(bp_hbm.at[idx_v.at[u, 0]], gp_v.at[u], sems.at[u, 0])
        pltpu.async_copy(az_hbm.at[pl.ds(r0, CHUNK)], az_v.at[u], sems.at[u, 1])

    def waits(t):
        u = t % 2
        pltpu.make_async_copy(bp_hbm.at[pl.ds(0, CHUNK)], gp_v.at[u], sems.at[u, 0]).wait()
        pltpu.make_async_copy(az_hbm.at[pl.ds(0, CHUNK)], az_v.at[u], sems.at[u, 1]).wait()

    def compute(t):
        u = t % 2

        def body(r, carry):
            db = jnp.zeros((16,), jnp.float32)
            ngp = jnp.zeros((16,), jnp.float32)
            naz = jnp.zeros((16,), jnp.float32)
            for d in range(LAN):
                sl = pl.ds(16 * d, 16)
                az = az_v[u, r, sl]
                gp = gp_v[u, r, sl]
                db = db + gp * az
                ngp = ngp + gp * gp
                naz = naz + az * az
            o_v[r, pl.ds(0, 16)] = db
            o_v[r, pl.ds(16, 16)] = ngp
            o_v[r, pl.ds(32, 16)] = naz
            return carry

        lax.fori_loop(0, CHUNK, body, 0, unroll=4)

    copies(0, wid)
    for t in range(n_t):
        g = wid + NW * t

        @pl.when(g < NCHUNK)
        def _(t=t):
            waits(t)

        if t + 1 < n_t:
            g2 = wid + NW * (t + 1)

            @pl.when(g2 < NCHUNK)
            def _(t=t, g2=g2):
                copies(t + 1, g2)

        @pl.when(g < NCHUNK)
        def _(t=t, g=g):
            compute(t)
            r0 = pl.multiple_of(g * CHUNK, 8)
            pltpu.sync_copy(o_v, out_hbm.at[pl.ds(r0, CHUNK)])


def _tc_onehot_body(idx_ref, m_ref, ap_ref, bz_ref, sa_ref, acc_ref):
    b = pl.program_id(0)
    jl = idx_ref[0] - b * NB
    m = m_ref[0]
    ap = ap_ref[0]
    bz = bz_ref[0]

    def nrm_rows(x):
        n = jnp.sqrt(jnp.sum(x * x, axis=1, keepdims=True))
        return x / jnp.maximum(n, EPS)

    nap = nrm_rows(ap)
    nbz = nrm_rows(bz)
    rowj = lax.broadcasted_iota(jnp.int32, (NB, NA), 0)
    ot = ((rowj == jnp.broadcast_to(jl, (NB, NA)))
          & (jnp.broadcast_to(m, (NB, NA)) >= 0.5)).astype(jnp.bfloat16)
    hi = nap.astype(jnp.bfloat16)
    lo = (nap - hi.astype(jnp.float32)).astype(jnp.bfloat16)
    dn = (((1,), (0,)), ((), ()))
    w = (lax.dot_general(ot, hi, dn, preferred_element_type=jnp.float32)
         + lax.dot_general(ot, lo, dn, preferred_element_type=jnp.float32))
    sa_b = jnp.sum(w * nbz)

    @pl.when(b == 0)
    def _():
        acc_ref[0] = 0.0

    acc_ref[0] = acc_ref[0] + sa_b

    @pl.when(b == B - 1)
    def _():
        sa_ref[0, 0] = acc_ref[0]


def _tc_final_body(sa_ref, o_ref, m_ref, loss_ref, c_ref):
    o = o_ref[...]
    ei = (lax.broadcasted_iota(jnp.int32, (64, 8), 0) // 16 ==
          lax.broadcasted_iota(jnp.int32, (64, 8), 1)).astype(jnp.float32)
    r = lax.dot_general(ei, o, (((0,), (1,)), ((), ())),
                        precision=lax.Precision.HIGHEST,
                        preferred_element_type=jnp.float32)
    db = r[0:1, :]
    ngp = r[1:2, :]
    naz = r[2:3, :]

    def nrm(x):
        return jnp.maximum(jnp.sqrt(x), EPS)

    pb = db / (nrm(ngp) * nrm(naz))
    sb = jnp.float32(0.0)
    cnt = jnp.float32(0.0)
    for b in range(B):
        m_b = m_ref[b]
        sb = sb + jnp.sum(m_b * pb[:, b * NA:(b + 1) * NA])
        cnt_b = jnp.sum(m_b)
        c_ref[b, 0] = cnt_b
        cnt = cnt + cnt_b
    denom = jnp.maximum(cnt, 1.0)
    total = sb + sa_ref[0, 0]
    loss_ref[...] = jnp.broadcast_to(-total / (2.0 * denom), (8, 128))


def kernel(feat_a_p, feat_a_z, feat_b_p, feat_b_z, iou, iou_threshold):
    thr = jnp.asarray(iou_threshold, jnp.float32).reshape(1)

    flat_idx, mask = pl.pallas_call(
        _tc_argmax_body,
        grid=(B,),
        in_specs=[
            pl.BlockSpec(memory_space=pltpu.SMEM),
            pl.BlockSpec((1, NA, NB), lambda b: (b, 0, 0)),
        ],
        out_specs=[
            pl.BlockSpec((1, 1, NA), lambda b: (b, 0, 0)),
            pl.BlockSpec((1, 1, NA), lambda b: (b, 0, 0)),
        ],
        out_shape=[
            jax.ShapeDtypeStruct((B, 1, NA), jnp.int32),
            jax.ShapeDtypeStruct((B, 1, NA), jnp.float32),
        ],
    )(thr, iou)

    idx3d = flat_idx.reshape(NCHUNK, 1, CHUNK)

    mesh = plsc.VectorSubcoreMesh(core_axis_name="c", subcore_axis_name="s")
    sc_dots = functools.partial(
        pl.kernel,
        out_type=jax.ShapeDtypeStruct((N, 64), jnp.float32),
        mesh=mesh,
        scratch_types=[
            pltpu.VMEM((2, 1, CHUNK), jnp.int32),
            pltpu.VMEM((2, CHUNK, D), jnp.float32),
            pltpu.VMEM((2, CHUNK, D), jnp.float32),
            pltpu.VMEM((CHUNK, 64), jnp.float32),
            pltpu.SemaphoreType.DMA((2, 2)),
        ],
    )(_sc_dot_body)
    packed = sc_dots(
        feat_a_z.reshape(N, D),
        feat_b_p.reshape(B * NB, D),
        idx3d,
    )

    sa_o = pl.pallas_call(
        _tc_onehot_body,
        grid=(B,),
        in_specs=[
            pl.BlockSpec((1, 1, NA), lambda b: (b, 0, 0)),
            pl.BlockSpec((1, 1, NA), lambda b: (b, 0, 0)),
            pl.BlockSpec((1, NA, D), lambda b: (b, 0, 0)),
            pl.BlockSpec((1, NB, D), lambda b: (b, 0, 0)),
        ],
        out_specs=pl.BlockSpec(memory_space=pltpu.SMEM),
        out_shape=jax.ShapeDtypeStruct((1, 1), jnp.float32),
        scratch_shapes=[pltpu.SMEM((1,), jnp.float32)],
    )(flat_idx, mask, feat_a_p, feat_b_z)

    loss_o, cnt_o = pl.pallas_call(
        _tc_final_body,
        grid=(1,),
        in_specs=[
            pl.BlockSpec(memory_space=pltpu.SMEM),
            pl.BlockSpec((N, 64), lambda i: (0, 0)),
            pl.BlockSpec((B, 1, NA), lambda i: (0, 0, 0)),
        ],
        out_specs=[
            pl.BlockSpec((8, 128), lambda i: (0, 0)),
            pl.BlockSpec(memory_space=pltpu.SMEM),
        ],
        out_shape=[
            jax.ShapeDtypeStruct((8, 128), jnp.float32),
            jax.ShapeDtypeStruct((B, 1), jnp.float32),
        ],
    )(sa_o, packed, mask)

    return (loss_o[0, 0], cnt_o[:, 0])
